# baseline (speedup 1.0000x reference)
import functools

import jax
import jax.numpy as jnp
from jax import lax
from jax.experimental import pallas as pl
from jax.experimental.pallas import tpu as pltpu
from jax.experimental.pallas import tpu_sc as plsc

N = 10000
E = 320000
NF = 128
EF = 16
H = 300
HP = 320
HH = 160
NG = 256
ODIM = 300

NQ = 4
QW = HP // NQ
QP = 128

NS = 16
BE = 80
ET = E // NS
NBLK = ET // BE
NP = 10112
RPT = NP // NS

GB = 1
GE = GB * BE
CB = 5
CE = CB * BE
NCH = NBLK // CB
NGR = CB // GB

f32 = jnp.float32
bf16 = jnp.bfloat16


def _pad2(a, r, c):
    return jnp.pad(a, ((0, r - a.shape[0]), (0, c - a.shape[1])))


def _bdot(a, w):
    return jnp.dot(a.astype(bf16), w, preferred_element_type=f32)


def _node_enc_body(x_ref, w0, w1, w2, b0, b1, b2, out_ref):
    x = x_ref[...]
    h = jnp.maximum(_bdot(x, w0[...]) + b0[...], 0.0)
    g0 = jnp.maximum(_bdot(h, w1[0]) + b1[0], 0.0)
    g1 = jnp.maximum(_bdot(h, w1[1]) + b1[1], 0.0)
    for q in range(4):
        t = _bdot(g0, w2[0, q]) + _bdot(g1, w2[1, q]) + b2[q]
        out_ref[q] = jnp.maximum(t, 0.0)


_BEE = 5000


def _edge_enc_body(ea_ref, w0, w1, w2, b0, b1, b2, out_ref):
    a = ea_ref[...].astype(bf16)
    h = jnp.maximum(jnp.dot(a, w0[...], preferred_element_type=f32)
                    + b0[...], 0.0).astype(bf16)
    g0 = jnp.maximum(jnp.dot(h, w1[0], preferred_element_type=f32)
                     + b1[0], 0.0).astype(bf16)
    g1 = jnp.maximum(jnp.dot(h, w1[1], preferred_element_type=f32)
                     + b1[1], 0.0).astype(bf16)
    for q in range(4):
        t = (jnp.dot(g0, w2[0, q], preferred_element_type=f32)
             + jnp.dot(g1, w2[1, q], preferred_element_type=f32) + b2[q])
        out_ref[q] = jnp.maximum(t, 0.0)


_RB = 2000


def _layer_mm_body(h_ref, a_ref, w4, bv, out_ref):
    xq = [h_ref[q] + a_ref[q] for q in range(4)]
    for qo in range(4):
        t = bv[0, qo]
        for qi in range(4):
            t = t + _bdot(xq[qi], w4[qi, qo])
        out_ref[qo] = t


def _layer_bn_body(t_ref, bv, out_ref, *, do_relu):
    t = t_ref[0]
    mu = jnp.mean(t, axis=0, keepdims=True)
    var = jnp.mean(jnp.square(t - mu), axis=0, keepdims=True)
    y = (t - mu) * lax.rsqrt(var + 1e-5) * bv[1, 0] + bv[2, 0]
    if do_relu:
        y = jnp.maximum(y, 0.0)
    out_ref[0] = y


def _pool_body(h_ref, b_ref, o0, ob0, o1, ob1, z_ref):
    gid = lax.broadcasted_iota(jnp.int32, (NG, N), 0)
    oh = (gid == b_ref[...]).astype(bf16)
    zs = []
    for q in range(4):
        h = h_ref[q]
        hh = h.astype(bf16)
        hl = (h - hh.astype(f32)).astype(bf16)
        zs.append(jnp.dot(oh, hh, preferred_element_type=f32)
                  + jnp.dot(oh, hl, preferred_element_type=f32))
    ts = []
    for nc in range(2):
        t = ob0[nc]
        for q in range(4):
            t = t + _bdot(zs[q], o0[q, nc])
        ts.append(jnp.maximum(t, 0.0))
    z_ref[...] = _bdot(ts[0], o1[0]) + _bdot(ts[1], o1[1]) + ob1[...]


def _sc_msg_body(h_hbm, e_hbm, src_hbm, dst_hbm, w_hbm, out_hbm,
                 src_c, dst_c, w_c, h_v, e_v, agg_sh, sem0, sem1):
    c = lax.axis_index("c")
    s = lax.axis_index("s")
    sems = (sem0, sem1)
    row0 = s * RPT

    for qp in range(2):
        qi = c * 2 + qp

        def _z(i, _):
            r = i // (QP // 16)
            k = i - r * (QP // 16)
            h_v[0, r, pl.ds(k * 16, 16)] = jnp.zeros((16,), f32)
            return 0
        lax.fori_loop(0, GE * (QP // 16), _z, 0)
        for j in range(RPT // GE):
            pltpu.sync_copy(h_v.at[0], agg_sh.at[pl.ds(row0 + j * GE, GE)])
        rem = RPT - (RPT // GE) * GE
        pltpu.sync_copy(h_v.at[0, pl.ds(0, rem)],
                        agg_sh.at[pl.ds(row0 + RPT - rem, rem)])

        plsc.subcore_barrier()
        off = qi * N

        def _chunk(ch, _):
            cbase = s * ET + ch * CE
            pltpu.sync_copy(src_hbm.at[pl.ds(cbase, CE)], src_c)
            pltpu.sync_copy(dst_hbm.at[s * NCH + ch], dst_c)
            pltpu.sync_copy(w_hbm.at[pl.ds(cbase, CE)], w_c)

            def _ofs(k, _):
                sl = pl.ds(k * 16, 16)
                src_c[sl] = src_c[sl] + off
                return 0
            lax.fori_loop(0, CE // 16, _ofs, 0)

            def _fire(g):
                slot = g % 2
                cps = [pltpu.async_copy(
                    h_hbm.at[src_c.at[pl.ds(g * GE, GE)]],
                    h_v.at[slot], sems[slot])]
                cps.append(pltpu.async_copy(
                    e_hbm.at[pl.ds(qi * E + cbase + g * GE, GE)],
                    e_v.at[slot], sems[slot]))
                return cps

            pend = {0: _fire(0), 1: None}
            for g in range(NGR):
                slot = g % 2
                if g + 1 < NGR:
                    pend[1 - slot] = _fire(g + 1)
                for cp in pend[slot]:
                    cp.wait()

                def _edge(jj, _):
                    for u in range(2):
                        j = jj * 2 + u
                        je = g * GE + j
                        jg = je // 16
                        jl = je - jg * 16
                        wv = w_c[pl.ds(jg * 16, 16)]
                        w16 = wv.at[jnp.full((16,), jl, jnp.int32)].get(
                            mode="promise_in_bounds")
                        for k in range(QW // 16):
                            sl = pl.ds(k * 16, 16)
                            e_v[slot, j, sl] = (
                                jnp.maximum(h_v[slot, j, sl]
                                            + e_v[slot, j, sl], 0.0) * w16)
                    return 0
                lax.fori_loop(0, GE // 2, _edge, 0)

                pltpu.async_copy(e_v.at[slot], agg_sh.at[dst_c.at[g]],
                                 sems[slot], add=True).wait()
            return 0
        lax.fori_loop(0, NCH, _chunk, 0)

        plsc.subcore_barrier()
        pltpu.sync_copy(agg_sh.at[pl.ds(row0, RPT)],
                        out_hbm.at[pl.ds(qi * NP + row0, RPT)])
        plsc.subcore_barrier()


def _make_sc_msg():
    mesh = plsc.VectorSubcoreMesh(core_axis_name="c", subcore_axis_name="s",
                                  num_cores=2, num_subcores=NS)
    return pl.kernel(
        _sc_msg_body,
        out_type=jax.ShapeDtypeStruct((NQ * NP, QP), f32),
        mesh=mesh,
        scratch_types=[
            pltpu.VMEM((CE,), jnp.int32),
            pltpu.VMEM((8, BE), jnp.int32),
            pltpu.VMEM((CE,), f32),
            pltpu.VMEM((2, GE, QP), f32),
            pltpu.VMEM((2, GE, QP), f32),
            pltpu.VMEM_SHARED((NP, QP), f32),
            pltpu.SemaphoreType.DMA,
            pltpu.SemaphoreType.DMA,
        ],
    )


def _enc_weights(Ws, bs, in_dim):
    w0 = Ws[0].astype(bf16)
    w1p = _pad2(Ws[1], Ws[1].shape[0], HP)
    w1 = jnp.stack([w1p[:, :HH], w1p[:, HH:]]).astype(bf16)
    w2p = _pad2(Ws[2], HP, HP)
    w2 = jnp.stack([
        jnp.stack([_pad2(w2p[kc * HH:(kc + 1) * HH,
                             q * QW:(q + 1) * QW], HH, QP)
                   for q in range(4)])
        for kc in range(2)
    ]).astype(bf16)
    b0 = bs[0].reshape(1, in_dim)
    b1p = jnp.pad(bs[1], (0, HP - bs[1].shape[0]))
    b1 = jnp.stack([b1p[:HH].reshape(1, HH), b1p[HH:].reshape(1, HH)])
    b2p = jnp.pad(bs[2], (0, HP - bs[2].shape[0]))
    b2 = jnp.stack([jnp.pad(b2p[q * QW:(q + 1) * QW],
                            (0, QP - QW)).reshape(1, QP)
                    for q in range(4)])
    return (w0, w1, w2, b0, b1, b2)


def _blk44(w):
    wp = _pad2(w, HP, HP)
    return jnp.stack([
        jnp.stack([_pad2(wp[qi * QW:(qi + 1) * QW, qo * QW:(qo + 1) * QW],
                         QP, QP)
                   for qo in range(4)])
        for qi in range(4)
    ]).astype(bf16)


def _blk42o(w):
    wp = _pad2(w, HP, HP)
    return jnp.stack([
        jnp.stack([_pad2(wp[q * QW:(q + 1) * QW, :HH], QP, HH),
                   _pad2(wp[q * QW:(q + 1) * QW, HH:], QP, HH)])
        for q in range(4)
    ]).astype(bf16)


def _halves2(v):
    vp = jnp.pad(v, (0, HP - v.shape[0]))
    return jnp.stack([vp[:HH].reshape(1, HH), vp[HH:].reshape(1, HH)])


def _quartersp(v):
    vp = jnp.pad(v, (0, HP - v.shape[0]))
    return jnp.stack([jnp.pad(vp[q * QW:(q + 1) * QW],
                              (0, QP - QW)).reshape(1, QP)
                      for q in range(4)])


def kernel(batch, x, edge_index, edge_attr, edge_weight, params):
    wspec = lambda a: pl.BlockSpec(a.shape, lambda i, nd=a.ndim: (0,) * nd)

    nw = _enc_weights(params['atom_W'], params['atom_b'], NF)
    h_st = pl.pallas_call(
        _node_enc_body,
        grid=(N // _RB,),
        in_specs=[pl.BlockSpec((_RB, NF), lambda r: (r, 0))]
        + [wspec(a) for a in nw],
        out_specs=pl.BlockSpec((4, _RB, QP), lambda r: (0, r, 0)),
        out_shape=jax.ShapeDtypeStruct((4, N, QP), f32),
    )(x, *nw)

    ew = _enc_weights(params['bond_W'], params['bond_b'], EF)
    e_st = pl.pallas_call(
        _edge_enc_body,
        grid=(E // _BEE,),
        in_specs=[pl.BlockSpec((_BEE, EF), lambda i: (i, 0))]
        + [wspec(a) for a in ew],
        out_specs=pl.BlockSpec((4, _BEE, QP), lambda i: (0, i, 0)),
        out_shape=jax.ShapeDtypeStruct((4, E, QP), f32),
    )(edge_attr, *ew)

    sc_msg = _make_sc_msg()
    src = edge_index[0].astype(jnp.int32)
    dst = jnp.pad(edge_index[1].astype(jnp.int32).reshape(
        NS * NCH, CB, BE), ((0, 0), (0, 8 - CB), (0, 0)))
    wvec = edge_weight.reshape(E).astype(f32)
    e_flat = e_st.reshape(NQ * E, QP)

    hcur = h_st
    for i in range(3):
        agg = sc_msg(hcur.reshape(NQ * N, QP), e_flat, src, dst,
                     wvec).reshape(NQ, NP, QP)[:, :N]
        w4 = _blk44(params['conv_W'][i])
        bvq = jnp.stack([_quartersp(params['conv_b'][i]),
                         _quartersp(params['bn_g'][i]),
                         _quartersp(params['bn_b'][i])])
        t = pl.pallas_call(
            _layer_mm_body,
            grid=(N // _RB,),
            in_specs=[
                pl.BlockSpec((4, _RB, QP), lambda r: (0, r, 0)),
                pl.BlockSpec((4, _RB, QP), lambda r: (0, r, 0)),
                wspec(w4), wspec(bvq),
            ],
            out_specs=pl.BlockSpec((4, _RB, QP), lambda r: (0, r, 0)),
            out_shape=jax.ShapeDtypeStruct((4, N, QP), f32),
        )(hcur, agg, w4, bvq)
        hcur = pl.pallas_call(
            functools.partial(_layer_bn_body, do_relu=(i != 2)),
            grid=(4,),
            in_specs=[
                pl.BlockSpec((1, N, QP), lambda q: (q, 0, 0)),
                pl.BlockSpec((3, 1, 1, QP), lambda q: (0, q, 0, 0)),
            ],
            out_specs=pl.BlockSpec((1, N, QP), lambda q: (q, 0, 0)),
            out_shape=jax.ShapeDtypeStruct((4, N, QP), f32),
        )(t, bvq)

    o0 = _blk42o(params['out_W'][0])
    o1p = _pad2(params['out_W'][1], HP, HP)
    o1 = jnp.stack([o1p[:HH], o1p[HH:]]).astype(bf16)
    ob0 = _halves2(params['out_b'][0])
    ob1 = jnp.pad(params['out_b'][1], (0, HP - ODIM)).reshape(1, HP)
    zfull = pl.pallas_call(
        _pool_body,
        out_shape=jax.ShapeDtypeStruct((NG, HP), f32),
    )(hcur, batch.astype(jnp.int32).reshape(1, N), o0, ob0, o1, ob1)

    z = zfull[:, :ODIM]
    node_emb = jnp.concatenate([hcur[q][:, :QW] for q in range(4)],
                               axis=1)[:, :H]
    return z, node_emb
